# baseline (speedup 1.0000x reference)
import dataclasses
import functools

import jax
import jax.numpy as jnp
from jax import lax
from jax.experimental import pallas as pl
from jax.experimental.pallas import tpu as pltpu
from jax.experimental.pallas import tpu_sc as plsc

N = 10000
E = 320000
D = 128
EPS = 1e-5

NC = 2
NS = 16
NW = NC * NS
CHUNK = 128
CPW = 80
HALF = CPW // 2
PADE = NW * CPW * CHUNK
NCHUNKS = PADE // CHUNK
OWN = 320
NP = OWN * NW
ACCP = 336
SCANB = 8
NSCAN = NCHUNKS // SCANB
MCAP = 16544
DRAIN_T = MCAP - 2 * SCANB * CHUNK - 144
STRIPE = NP // NS


def _compiler_params():
    cp = pltpu.CompilerParams()
    if "needs_layout_passes" in pltpu.CompilerParams.__dataclass_fields__:
        cp = dataclasses.replace(cp, needs_layout_passes=False)
    return cp


def _mesh():
    return plsc.VectorSubcoreMesh(core_axis_name="c", subcore_axis_name="s",
                                  num_cores=NC, num_subcores=NS)


def _sc_sum_body(x_hbm, src_hbm, dst_hbm, ssum_out,
                 sidx_v, didx_v, rowsA, rowsB, ssum_sh, semA, semB):
    c = lax.axis_index("c")
    s = lax.axis_index("s")
    wid = s * NC + c
    zvec = jnp.zeros((16,), jnp.float32)

    with jax.named_scope("sum_zero"):
        @pl.loop(0, CHUNK)
        def _(i):
            for j in range(8):
                rowsA[i, pl.ds(j * 16, 16)] = zvec

        for k in range(STRIPE // CHUNK):
            pltpu.sync_copy(rowsA,
                            ssum_sh.at[pl.ds(s * STRIPE + k * CHUNK, CHUNK)])
        plsc.subcore_barrier()

    for h in range(2):
      with jax.named_scope(f"sum_half{h}"):
        base = wid * CPW + h * HALF
        pltpu.sync_copy(src_hbm.at[pl.ds(base, HALF)], sidx_v)
        pltpu.sync_copy(dst_hbm.at[pl.ds(base, HALF)], didx_v)
        pltpu.async_copy(x_hbm.at[sidx_v.at[0]], rowsA, semA)

        @pl.loop(0, HALF // 2)
        def _(k2):
            kA = 2 * k2
            kB = kA + 1
            pltpu.make_async_copy(x_hbm.at[sidx_v.at[kA]], rowsA, semA).wait()
            pltpu.async_copy(x_hbm.at[sidx_v.at[kB]], rowsB, semB)
            pltpu.sync_copy(rowsA, ssum_sh.at[didx_v.at[kA]], add=True)
            pltpu.make_async_copy(x_hbm.at[sidx_v.at[kB]], rowsB, semB).wait()

            @pl.when(kB + 1 < HALF)
            def _():
                pltpu.async_copy(x_hbm.at[sidx_v.at[kB + 1]], rowsA, semA)

            pltpu.sync_copy(rowsB, ssum_sh.at[didx_v.at[kB]], add=True)

    with jax.named_scope("sum_out"):
        plsc.subcore_barrier()
        sl = pl.ds(s * STRIPE, STRIPE)
        pltpu.sync_copy(ssum_sh.at[sl], ssum_out.at[c].at[sl])


def _sc_max_body(hp_hbm, src_hbm, dst_hbm, hmax_out, deg_out,
                 rowsA, rowsB, acc_v, deg_v, msrc_v, mdst_v,
                 dscanA, sscanA, dscanB, sscanB, spst, smdst,
                 semA, semB, semG):
    c = lax.axis_index("c")
    s = lax.axis_index("s")
    wid = s * NC + c
    zvec = jnp.zeros((16,), jnp.float32)
    zivec = jnp.zeros((16,), jnp.int32)
    dummyvec = jnp.full((16,), OWN, jnp.int32)
    onesf = jnp.full((16,), 1.0, jnp.float32)

    @pl.loop(0, ACCP)
    def _(i):
        for j in range(8):
            acc_v[i, pl.ds(j * 16, 16)] = zvec

    @pl.loop(0, ACCP // 16)
    def _(i):
        deg_v[pl.ds(i * 16, 16)] = zvec

    lo = wid * OWN

    def stage_dsts(b, smdst):
        pltpu.sync_copy(mdst_v.at[pl.ds(b * CHUNK, CHUNK)], spst.at[s])
        pltpu.sync_copy(spst.at[s], smdst)

    def accum_batch(rows, smdst):
        @pl.loop(0, CHUNK, unroll=2)
        def _(i):
            d = smdst[i]
            for j in range(8):
                vsl = pl.ds(j * 16, 16)
                acc_v[d, vsl] = jnp.maximum(acc_v[d, vsl], rows[i, vsl])

    def drain(mc):
        for i in range(8):
            msrc_v[pl.ds(mc + 16 * i, 16)] = zivec
            mdst_v[pl.ds(mc + 16 * i, 16)] = dummyvec
        nb = (mc + CHUNK - 1) // CHUNK

        @pl.when(nb > 0)
        def _():
            pltpu.async_copy(hp_hbm.at[msrc_v.at[pl.ds(0, CHUNK)]],
                             rowsA, semG)

        def batch2_body(b2, _):
            bA = 2 * b2
            bB = bA + 1
            stage_dsts(bA, smdst)
            pltpu.make_async_copy(hp_hbm.at[msrc_v.at[pl.ds(0, CHUNK)]],
                                  rowsA, semG).wait()

            @pl.when(bB < nb)
            def _():
                pltpu.async_copy(
                    hp_hbm.at[msrc_v.at[pl.ds(bB * CHUNK, CHUNK)]],
                    rowsB, semG)

            accum_batch(rowsA, smdst)

            @pl.when(bB < nb)
            def _():
                stage_dsts(bB, smdst)
                pltpu.make_async_copy(
                    hp_hbm.at[msrc_v.at[pl.ds(0, CHUNK)]],
                    rowsB, semG).wait()

                @pl.when(bB + 1 < nb)
                def _():
                    pltpu.async_copy(
                        hp_hbm.at[msrc_v.at[pl.ds((bB + 1) * CHUNK, CHUNK)]],
                        rowsA, semG)

                accum_batch(rowsB, smdst)
            return 0

        lax.fori_loop(0, (nb + 1) // 2, batch2_body, 0)

    def filt(dscan, sscan, mc):
        def row_body(r, mc_):
            for j in range(8):
                vsl = pl.ds(j * 16, 16)
                dvec = dscan[r, vsl]
                svec = sscan[r, vsl]
                dloc = dvec - lo
                m = (dloc >= 0) & (dloc < OWN) & (dvec < N)
                plsc.addupdate_scatter(deg_v, [dloc], onesf, mask=m)
                plsc.store_compressed(msrc_v.at[pl.ds(mc_, 16)], svec, mask=m)
                plsc.store_compressed(mdst_v.at[pl.ds(mc_, 16)], dloc, mask=m)
                pc = plsc.all_reduce_population_count(m)
                mc_ = mc_ + pc[0]
            return mc_
        return lax.fori_loop(0, SCANB, row_body, mc)

    def maybe_drain(mc):
        def do_drain(m):
            drain(m)
            return 0
        return lax.cond(mc > DRAIN_T, do_drain, lambda m: m, mc)

    with jax.named_scope("max_scan_start"):
        pltpu.async_copy(dst_hbm.at[pl.ds(0, SCANB)], dscanA, semA)
        pltpu.async_copy(src_hbm.at[pl.ds(0, SCANB)], sscanA, semA)

    def block2_body(t2, mc):
        tA = 2 * t2
        tB = tA + 1
        pltpu.make_async_copy(dst_hbm.at[pl.ds(0, SCANB)], dscanA, semA).wait()
        pltpu.make_async_copy(src_hbm.at[pl.ds(0, SCANB)], sscanA, semA).wait()
        pltpu.async_copy(dst_hbm.at[pl.ds(tB * SCANB, SCANB)], dscanB, semB)
        pltpu.async_copy(src_hbm.at[pl.ds(tB * SCANB, SCANB)], sscanB, semB)
        mc = filt(dscanA, sscanA, mc)
        pltpu.make_async_copy(dst_hbm.at[pl.ds(0, SCANB)], dscanB, semB).wait()
        pltpu.make_async_copy(src_hbm.at[pl.ds(0, SCANB)], sscanB, semB).wait()

        @pl.when(tB + 1 < NSCAN)
        def _():
            pltpu.async_copy(dst_hbm.at[pl.ds((tB + 1) * SCANB, SCANB)],
                             dscanA, semA)
            pltpu.async_copy(src_hbm.at[pl.ds((tB + 1) * SCANB, SCANB)],
                             sscanA, semA)

        mc = filt(dscanB, sscanB, mc)
        return maybe_drain(mc)

    with jax.named_scope("max_scan"):
        mcnt = lax.fori_loop(0, NSCAN // 2, block2_body, 0)
    with jax.named_scope("max_drain"):
        drain(mcnt)

    with jax.named_scope("max_out"):
        pltpu.sync_copy(acc_v.at[pl.ds(0, OWN)],
                        hmax_out.at[pl.ds(wid * OWN, OWN)])
        pltpu.sync_copy(deg_v.at[pl.ds(0, OWN)],
                        deg_out.at[pl.ds(wid * OWN, OWN)])


def _sc_sum(x, src2d, dst2d):
    return pl.kernel(
        _sc_sum_body,
        out_type=jax.ShapeDtypeStruct((NC, NP, D), jnp.float32),
        mesh=_mesh(),
        scratch_types=[
            pltpu.VMEM((HALF, CHUNK), jnp.int32),
            pltpu.VMEM((HALF, CHUNK), jnp.int32),
            pltpu.VMEM((CHUNK, D), jnp.float32),
            pltpu.VMEM((CHUNK, D), jnp.float32),
            pltpu.VMEM_SHARED((NP, D), jnp.float32),
            pltpu.SemaphoreType.DMA,
            pltpu.SemaphoreType.DMA,
        ],
        compiler_params=_compiler_params(),
    )(x, src2d, dst2d)


def _sc_max(hp, src2d, dst2d):
    return pl.kernel(
        _sc_max_body,
        out_type=[
            jax.ShapeDtypeStruct((NP, D), jnp.float32),
            jax.ShapeDtypeStruct((NP,), jnp.float32),
        ],
        mesh=_mesh(),
        scratch_types=[
            pltpu.VMEM((CHUNK, D), jnp.float32),
            pltpu.VMEM((CHUNK, D), jnp.float32),
            pltpu.VMEM((ACCP, D), jnp.float32),
            pltpu.VMEM((ACCP,), jnp.float32),
            pltpu.VMEM((MCAP,), jnp.int32),
            pltpu.VMEM((MCAP,), jnp.int32),
            pltpu.VMEM((SCANB, CHUNK), jnp.int32),
            pltpu.VMEM((SCANB, CHUNK), jnp.int32),
            pltpu.VMEM((SCANB, CHUNK), jnp.int32),
            pltpu.VMEM((SCANB, CHUNK), jnp.int32),
            pltpu.VMEM_SHARED((NS, CHUNK), jnp.int32),
            pltpu.SMEM((CHUNK,), jnp.int32),
            pltpu.SemaphoreType.DMA,
            pltpu.SemaphoreType.DMA,
            pltpu.SemaphoreType.DMA,
        ],
        compiler_params=_compiler_params(),
    )(hp, src2d, dst2d)


def _pool_mlp_body(x_ref, wp_ref, bp_ref, hp_ref):
    hp_ref[...] = jnp.maximum(
        jnp.dot(x_ref[...], wp_ref[...].T, preferred_element_type=jnp.float32)
        + bp_ref[...],
        0.0,
    )


def _final_body(x_ref, ssum_ref, deg_ref, hmax_ref, ws1_ref, wn1_ref,
                ws2_ref, wn2_ref, g1_ref, b1_ref, g2_ref, b2_ref, out_ref):
    x = x_ref[...]
    deg = jnp.maximum(deg_ref[...], 1.0)
    h_mean = (ssum_ref[0, :N] + ssum_ref[1, :N]) / deg
    z1 = (jnp.dot(x, ws1_ref[...].T, preferred_element_type=jnp.float32)
          + jnp.dot(h_mean, wn1_ref[...].T, preferred_element_type=jnp.float32))
    z2 = (jnp.dot(x, ws2_ref[...].T, preferred_element_type=jnp.float32)
          + jnp.dot(hmax_ref[:N], wn2_ref[...].T,
                    preferred_element_type=jnp.float32))

    def bn(z, g, b):
        mu = jnp.mean(z, axis=0, keepdims=True)
        var = jnp.mean((z - mu) ** 2, axis=0, keepdims=True)
        return (z - mu) * lax.rsqrt(var + EPS) * g + b

    t = bn(z1, g1_ref[...], b1_ref[...]) + bn(z2, g2_ref[...], b2_ref[...])
    out_ref[...] = jnp.where(t >= 0.0, t, 0.01 * t)


@functools.partial(jax.jit, static_argnums=())
def kernel(x, edge_index, W_self1, W_neigh1, W_pool, b_pool, W_self2,
           W_neigh2, gamma1, beta1, gamma2, beta2):
    hp = pl.pallas_call(
        _pool_mlp_body,
        out_shape=jax.ShapeDtypeStruct((N, D), jnp.float32),
    )(x, W_pool, b_pool.reshape(1, D))

    pad = PADE - E
    ar = jnp.arange(pad, dtype=jnp.int32)
    srcp = jnp.concatenate([edge_index[0], ar % N])
    dstp = jnp.concatenate([edge_index[1], N + ar % (NP - N)])
    src2d = srcp.reshape(NCHUNKS, CHUNK)
    dst2d = dstp.reshape(NCHUNKS, CHUNK)
    ssum_p = _sc_sum(x, src2d, dst2d)
    hmax_p, deg_p = _sc_max(hp, src2d, dst2d)

    out = pl.pallas_call(
        _final_body,
        out_shape=jax.ShapeDtypeStruct((N, D), jnp.float32),
    )(x, ssum_p, deg_p[:N].reshape(N, 1), hmax_p,
      W_self1, W_neigh1, W_self2, W_neigh2,
      gamma1.reshape(1, D), beta1.reshape(1, D),
      gamma2.reshape(1, D), beta2.reshape(1, D))
    return out


# speedup vs baseline: 1.0094x; 1.0094x over previous
import dataclasses
import functools

import jax
import jax.numpy as jnp
from jax import lax
from jax.experimental import pallas as pl
from jax.experimental.pallas import tpu as pltpu
from jax.experimental.pallas import tpu_sc as plsc

N = 10000
E = 320000
D = 128
EPS = 1e-5

NC = 2
NS = 16
NW = NC * NS
CHUNK = 128
CPW = 80
HALF = CPW // 2
PADE = NW * CPW * CHUNK
NCHUNKS = PADE // CHUNK
OWN = 320
NP = OWN * NW
ACCP = 336
SCANB = 8
NSCAN = NCHUNKS // SCANB
MCAP = 16544
DRAIN_T = MCAP - 2 * SCANB * CHUNK - 144
STRIPE = NP // NS


def _compiler_params():
    cp = pltpu.CompilerParams()
    if "needs_layout_passes" in pltpu.CompilerParams.__dataclass_fields__:
        cp = dataclasses.replace(cp, needs_layout_passes=False)
    return cp


def _mesh():
    return plsc.VectorSubcoreMesh(core_axis_name="c", subcore_axis_name="s",
                                  num_cores=NC, num_subcores=NS)


def _sc_sum_body(x_hbm, src_hbm, dst_hbm, ssum_out,
                 sidx_v, didx_v, rowsA, rowsB, ssum_sh, semA, semB):
    c = lax.axis_index("c")
    s = lax.axis_index("s")
    wid = s * NC + c
    zvec = jnp.zeros((16,), jnp.float32)

    with jax.named_scope("sum_zero"):
        @pl.loop(0, CHUNK)
        def _(i):
            for j in range(8):
                rowsA[i, pl.ds(j * 16, 16)] = zvec

        for k in range(STRIPE // CHUNK):
            pltpu.sync_copy(rowsA,
                            ssum_sh.at[pl.ds(s * STRIPE + k * CHUNK, CHUNK)])
        plsc.subcore_barrier()

    for h in range(2):
      with jax.named_scope(f"sum_half{h}"):
        base = wid * CPW + h * HALF
        pltpu.sync_copy(src_hbm.at[pl.ds(base, HALF)], sidx_v)
        pltpu.sync_copy(dst_hbm.at[pl.ds(base, HALF)], didx_v)
        pltpu.async_copy(x_hbm.at[sidx_v.at[0]], rowsA, semA)

        @pl.loop(0, HALF // 2)
        def _(k2):
            kA = 2 * k2
            kB = kA + 1
            pltpu.make_async_copy(x_hbm.at[sidx_v.at[kA]], rowsA, semA).wait()
            pltpu.async_copy(x_hbm.at[sidx_v.at[kB]], rowsB, semB)
            pltpu.sync_copy(rowsA, ssum_sh.at[didx_v.at[kA]], add=True)
            pltpu.make_async_copy(x_hbm.at[sidx_v.at[kB]], rowsB, semB).wait()

            @pl.when(kB + 1 < HALF)
            def _():
                pltpu.async_copy(x_hbm.at[sidx_v.at[kB + 1]], rowsA, semA)

            pltpu.sync_copy(rowsB, ssum_sh.at[didx_v.at[kB]], add=True)

    with jax.named_scope("sum_out"):
        plsc.subcore_barrier()
        sl = pl.ds(s * STRIPE, STRIPE)
        pltpu.sync_copy(ssum_sh.at[sl], ssum_out.at[c].at[sl])


def _sc_max_body(hp_hbm, src_hbm, dst_hbm, hmax_out, deg_out,
                 rowsA, rowsB, acc_v, deg_v, msrc_v, mdst_v,
                 dscanA, sscanA, dscanB, sscanB, semA, semB, semG):
    c = lax.axis_index("c")
    s = lax.axis_index("s")
    wid = s * NC + c
    zvec = jnp.zeros((16,), jnp.float32)
    zivec = jnp.zeros((16,), jnp.int32)
    dummyvec = jnp.full((16,), OWN, jnp.int32)
    onesf = jnp.full((16,), 1.0, jnp.float32)

    @pl.loop(0, ACCP)
    def _(i):
        for j in range(8):
            acc_v[i, pl.ds(j * 16, 16)] = zvec

    @pl.loop(0, ACCP // 16)
    def _(i):
        deg_v[pl.ds(i * 16, 16)] = zvec

    lo = wid * OWN

    def accum_batch(rows, b):
        def vec_body(v, b_):
            dvec = mdst_v[pl.ds(b_ * CHUNK + v * 16, 16)]
            for l in range(16):
                d = dvec[l]
                for j in range(8):
                    vsl = pl.ds(j * 16, 16)
                    acc_v[d, vsl] = jnp.maximum(acc_v[d, vsl],
                                                rows[v * 16 + l, vsl])
            return b_
        lax.fori_loop(0, CHUNK // 16, vec_body, b)

    def drain(mc):
        for i in range(8):
            msrc_v[pl.ds(mc + 16 * i, 16)] = zivec
            mdst_v[pl.ds(mc + 16 * i, 16)] = dummyvec
        nb = (mc + CHUNK - 1) // CHUNK

        @pl.when(nb > 0)
        def _():
            pltpu.async_copy(hp_hbm.at[msrc_v.at[pl.ds(0, CHUNK)]],
                             rowsA, semG)

        def batch2_body(b2, _):
            bA = 2 * b2
            bB = bA + 1
            pltpu.make_async_copy(hp_hbm.at[msrc_v.at[pl.ds(0, CHUNK)]],
                                  rowsA, semG).wait()

            @pl.when(bB < nb)
            def _():
                pltpu.async_copy(
                    hp_hbm.at[msrc_v.at[pl.ds(bB * CHUNK, CHUNK)]],
                    rowsB, semG)

            accum_batch(rowsA, bA)

            @pl.when(bB < nb)
            def _():
                pltpu.make_async_copy(
                    hp_hbm.at[msrc_v.at[pl.ds(0, CHUNK)]],
                    rowsB, semG).wait()

                @pl.when(bB + 1 < nb)
                def _():
                    pltpu.async_copy(
                        hp_hbm.at[msrc_v.at[pl.ds((bB + 1) * CHUNK, CHUNK)]],
                        rowsA, semG)

                accum_batch(rowsB, bB)
            return 0

        lax.fori_loop(0, (nb + 1) // 2, batch2_body, 0)

    def filt(dscan, sscan, mc):
        def row_body(r, mc_):
            for j in range(8):
                vsl = pl.ds(j * 16, 16)
                dvec = dscan[r, vsl]
                svec = sscan[r, vsl]
                dloc = dvec - lo
                m = (dloc >= 0) & (dloc < OWN) & (dvec < N)
                plsc.addupdate_scatter(deg_v, [dloc], onesf, mask=m)
                plsc.store_compressed(msrc_v.at[pl.ds(mc_, 16)], svec, mask=m)
                plsc.store_compressed(mdst_v.at[pl.ds(mc_, 16)], dloc, mask=m)
                pc = plsc.all_reduce_population_count(m)
                mc_ = mc_ + pc[0]
            return mc_
        return lax.fori_loop(0, SCANB, row_body, mc)

    def maybe_drain(mc):
        def do_drain(m):
            drain(m)
            return 0
        return lax.cond(mc > DRAIN_T, do_drain, lambda m: m, mc)

    with jax.named_scope("max_scan_start"):
        pltpu.async_copy(dst_hbm.at[pl.ds(0, SCANB)], dscanA, semA)
        pltpu.async_copy(src_hbm.at[pl.ds(0, SCANB)], sscanA, semA)

    def block2_body(t2, mc):
        tA = 2 * t2
        tB = tA + 1
        pltpu.make_async_copy(dst_hbm.at[pl.ds(0, SCANB)], dscanA, semA).wait()
        pltpu.make_async_copy(src_hbm.at[pl.ds(0, SCANB)], sscanA, semA).wait()
        pltpu.async_copy(dst_hbm.at[pl.ds(tB * SCANB, SCANB)], dscanB, semB)
        pltpu.async_copy(src_hbm.at[pl.ds(tB * SCANB, SCANB)], sscanB, semB)
        mc = filt(dscanA, sscanA, mc)
        pltpu.make_async_copy(dst_hbm.at[pl.ds(0, SCANB)], dscanB, semB).wait()
        pltpu.make_async_copy(src_hbm.at[pl.ds(0, SCANB)], sscanB, semB).wait()

        @pl.when(tB + 1 < NSCAN)
        def _():
            pltpu.async_copy(dst_hbm.at[pl.ds((tB + 1) * SCANB, SCANB)],
                             dscanA, semA)
            pltpu.async_copy(src_hbm.at[pl.ds((tB + 1) * SCANB, SCANB)],
                             sscanA, semA)

        mc = filt(dscanB, sscanB, mc)
        return maybe_drain(mc)

    with jax.named_scope("max_scan"):
        mcnt = lax.fori_loop(0, NSCAN // 2, block2_body, 0)
    with jax.named_scope("max_drain"):
        drain(mcnt)

    with jax.named_scope("max_out"):
        pltpu.sync_copy(acc_v.at[pl.ds(0, OWN)],
                        hmax_out.at[pl.ds(wid * OWN, OWN)])
        pltpu.sync_copy(deg_v.at[pl.ds(0, OWN)],
                        deg_out.at[pl.ds(wid * OWN, OWN)])


def _sc_sum(x, src2d, dst2d):
    return pl.kernel(
        _sc_sum_body,
        out_type=jax.ShapeDtypeStruct((NC, NP, D), jnp.float32),
        mesh=_mesh(),
        scratch_types=[
            pltpu.VMEM((HALF, CHUNK), jnp.int32),
            pltpu.VMEM((HALF, CHUNK), jnp.int32),
            pltpu.VMEM((CHUNK, D), jnp.float32),
            pltpu.VMEM((CHUNK, D), jnp.float32),
            pltpu.VMEM_SHARED((NP, D), jnp.float32),
            pltpu.SemaphoreType.DMA,
            pltpu.SemaphoreType.DMA,
        ],
        compiler_params=_compiler_params(),
    )(x, src2d, dst2d)


def _sc_max(hp, src2d, dst2d):
    return pl.kernel(
        _sc_max_body,
        out_type=[
            jax.ShapeDtypeStruct((NP, D), jnp.float32),
            jax.ShapeDtypeStruct((NP,), jnp.float32),
        ],
        mesh=_mesh(),
        scratch_types=[
            pltpu.VMEM((CHUNK, D), jnp.float32),
            pltpu.VMEM((CHUNK, D), jnp.float32),
            pltpu.VMEM((ACCP, D), jnp.float32),
            pltpu.VMEM((ACCP,), jnp.float32),
            pltpu.VMEM((MCAP,), jnp.int32),
            pltpu.VMEM((MCAP,), jnp.int32),
            pltpu.VMEM((SCANB, CHUNK), jnp.int32),
            pltpu.VMEM((SCANB, CHUNK), jnp.int32),
            pltpu.VMEM((SCANB, CHUNK), jnp.int32),
            pltpu.VMEM((SCANB, CHUNK), jnp.int32),
            pltpu.SemaphoreType.DMA,
            pltpu.SemaphoreType.DMA,
            pltpu.SemaphoreType.DMA,
        ],
        compiler_params=_compiler_params(),
    )(hp, src2d, dst2d)


def _pool_mlp_body(x_ref, wp_ref, bp_ref, hp_ref):
    hp_ref[...] = jnp.maximum(
        jnp.dot(x_ref[...], wp_ref[...].T, preferred_element_type=jnp.float32)
        + bp_ref[...],
        0.0,
    )


def _final_body(x_ref, ssum_ref, deg_ref, hmax_ref, ws1_ref, wn1_ref,
                ws2_ref, wn2_ref, g1_ref, b1_ref, g2_ref, b2_ref, out_ref):
    x = x_ref[...]
    deg = jnp.maximum(deg_ref[...], 1.0)
    h_mean = (ssum_ref[0, :N] + ssum_ref[1, :N]) / deg
    z1 = (jnp.dot(x, ws1_ref[...].T, preferred_element_type=jnp.float32)
          + jnp.dot(h_mean, wn1_ref[...].T, preferred_element_type=jnp.float32))
    z2 = (jnp.dot(x, ws2_ref[...].T, preferred_element_type=jnp.float32)
          + jnp.dot(hmax_ref[:N], wn2_ref[...].T,
                    preferred_element_type=jnp.float32))

    def bn(z, g, b):
        mu = jnp.mean(z, axis=0, keepdims=True)
        var = jnp.mean((z - mu) ** 2, axis=0, keepdims=True)
        return (z - mu) * lax.rsqrt(var + EPS) * g + b

    t = bn(z1, g1_ref[...], b1_ref[...]) + bn(z2, g2_ref[...], b2_ref[...])
    out_ref[...] = jnp.where(t >= 0.0, t, 0.01 * t)


@functools.partial(jax.jit, static_argnums=())
def kernel(x, edge_index, W_self1, W_neigh1, W_pool, b_pool, W_self2,
           W_neigh2, gamma1, beta1, gamma2, beta2):
    hp = pl.pallas_call(
        _pool_mlp_body,
        out_shape=jax.ShapeDtypeStruct((N, D), jnp.float32),
    )(x, W_pool, b_pool.reshape(1, D))

    pad = PADE - E
    ar = jnp.arange(pad, dtype=jnp.int32)
    srcp = jnp.concatenate([edge_index[0], ar % N])
    dstp = jnp.concatenate([edge_index[1], N + ar % (NP - N)])
    src2d = srcp.reshape(NCHUNKS, CHUNK)
    dst2d = dstp.reshape(NCHUNKS, CHUNK)
    ssum_p = _sc_sum(x, src2d, dst2d)
    hmax_p, deg_p = _sc_max(hp, src2d, dst2d)

    out = pl.pallas_call(
        _final_body,
        out_shape=jax.ShapeDtypeStruct((N, D), jnp.float32),
    )(x, ssum_p, deg_p[:N].reshape(N, 1), hmax_p,
      W_self1, W_neigh1, W_self2, W_neigh2,
      gamma1.reshape(1, D), beta1.reshape(1, D),
      gamma2.reshape(1, D), beta2.reshape(1, D))
    return out
